# baseline (speedup 1.0000x reference)
import math

import jax
import jax.numpy as jnp
from jax import lax
from jax.experimental import pallas as pl
from jax.experimental.pallas import tpu as pltpu
from jax.experimental.pallas import tpu_sc as plsc

H = 128
N = 10000
E = 320000
L = 131072
NC, NS = 2, 16
NW = NC * NS
NP = 10112
SL = NP // NS
RPS = 160
EP = RPS * NS * 128
CW = 16
CH = 16
KB = 4
EW = 64
RPS64 = RPS * 2
RW = (L // 128) // NW

_mesh = plsc.VectorSubcoreMesh(
    core_axis_name="c", subcore_axis_name="s", num_cores=NC, num_subcores=NS
)


def _segsum_body(
    xm, xu, smu, dmu, sum_, dum, zeros, out,
    sidx, didx, rows0, rows1, rows2, rows3, acc,
    gs0, gs1, gs2, gs3, ss0, ss1, ss2, ss3
):
    c = lax.axis_index("c")
    s = lax.axis_index("s")
    rows = [rows0, rows1, rows2, rows3]
    gsem = [gs0, gs1, gs2, gs3]
    ssem = [ss0, ss1, ss2, ss3]

    def run(x_hbm, s_hbm, d_hbm):
        pltpu.sync_copy(zeros.at[pl.ds(s * SL, SL)], acc.at[pl.ds(s * SL, SL)])
        plsc.subcore_barrier()

        def chunk(ch, carry):
            base = pl.multiple_of(s * RPS64 + ch * CH, 8)
            pltpu.sync_copy(s_hbm.at[pl.ds(base, CH)], sidx)
            pltpu.sync_copy(d_hbm.at[pl.ds(base, CH)], didx)
            gd = [None] * KB
            sd = [None] * KB
            for step in range(CH + 1):
                if step < CH:
                    sl = step % KB
                    if step >= KB:
                        sd[sl].wait()
                    gd[sl] = pltpu.async_copy(
                        x_hbm.at[sidx.at[step]], rows[sl], gsem[sl]
                    )
                prev = step - 1
                if prev >= 0:
                    sp = prev % KB
                    gd[sp].wait()
                    sd[sp] = pltpu.async_copy(
                        rows[sp], acc.at[didx.at[prev]], ssem[sp], add=True
                    )
            for k in range(KB):
                sd[k].wait()
            return carry

        lax.fori_loop(0, RPS64 // CH, chunk, 0)
        plsc.subcore_barrier()
        pltpu.sync_copy(acc.at[pl.ds(s * SL, SL)], out.at[c, pl.ds(s * SL, SL)])

    @pl.when(c == 0)
    def _():
        run(xm, smu, dmu)

    @pl.when(c == 1)
    def _():
        run(xu, sum_, dum)


_segsum = pl.kernel(
    _segsum_body,
    out_type=jax.ShapeDtypeStruct((NC, NP, H), jnp.float32),
    mesh=_mesh,
    scratch_types=[
        pltpu.VMEM((CH, EW), jnp.int32),
        pltpu.VMEM((CH, EW), jnp.int32),
        pltpu.VMEM((EW, H), jnp.float32),
        pltpu.VMEM((EW, H), jnp.float32),
        pltpu.VMEM((EW, H), jnp.float32),
        pltpu.VMEM((EW, H), jnp.float32),
        pltpu.VMEM_SHARED((NP, H), jnp.float32),
    ]
    + [pltpu.SemaphoreType.DMA] * 8,
)


def _counts_body(dmu, dum, ones, zeros, out, didx, onev, acc):
    c = lax.axis_index("c")
    s = lax.axis_index("s")
    pltpu.sync_copy(ones, onev)

    def run(d_hbm):
        pltpu.sync_copy(d_hbm.at[pl.ds(s * RPS, RPS)], didx)
        pltpu.sync_copy(zeros.at[pl.ds(s * SL, SL)], acc.at[pl.ds(s * SL, SL)])
        plsc.subcore_barrier()

        def body(r, carry):
            pltpu.sync_copy(onev, acc.at[didx.at[r]], add=True)
            return carry

        lax.fori_loop(0, RPS, body, 0)
        plsc.subcore_barrier()
        pltpu.sync_copy(acc.at[pl.ds(s * SL, SL)], out.at[c, pl.ds(s * SL, SL)])

    @pl.when(c == 0)
    def _():
        run(dmu)

    @pl.when(c == 1)
    def _():
        run(dum)


_counts = pl.kernel(
    _counts_body,
    out_type=jax.ShapeDtypeStruct((NC, NP, H), jnp.float32),
    mesh=_mesh,
    scratch_types=[
        pltpu.VMEM((RPS, 128), jnp.int32),
        pltpu.VMEM((128, H), jnp.float32),
        pltpu.VMEM_SHARED((NP, H), jnp.float32),
    ],
)


def _pairsum_body(pu, pm, e0, e1, out, idx0, idx1, buf, sem0, sem1):
    c = lax.axis_index("c")
    s = lax.axis_index("s")
    w = s * NC + c
    base = w * RW
    pltpu.sync_copy(e0.at[pl.ds(base, RW)], idx0)
    pltpu.sync_copy(e1.at[pl.ds(base, RW)], idx1)

    def body(r, carry):
        pltpu.async_copy(pu.at[idx0.at[r]], buf, sem0).wait()
        pltpu.async_copy(pm.at[idx1.at[r]], buf, sem1, add=True).wait()
        pltpu.sync_copy(buf, out.at[pl.ds((base + r) * 128, 128)])
        return carry

    lax.fori_loop(0, RW, body, 0)


_pairsum = pl.kernel(
    _pairsum_body,
    out_type=jax.ShapeDtypeStruct((L, H), jnp.float32),
    mesh=_mesh,
    scratch_types=[
        pltpu.VMEM((RW, 128), jnp.int32),
        pltpu.VMEM((RW, 128), jnp.int32),
        pltpu.VMEM((128, H), jnp.float32),
        pltpu.SemaphoreType.DMA,
        pltpu.SemaphoreType.DMA,
    ],
)


RB = 1000


def _layer_tc_body(su, sm, cu, cm, xu, xm, wum, wur, bu, wmm, wmr, bm, ou, om):
    mu = su[...] / jnp.maximum(cu[...][:, :1], 1.0)
    mm = sm[...] / jnp.maximum(cm[...][:, :1], 1.0)
    ou[...] = jax.nn.relu(
        jnp.dot(mu, wum[...], preferred_element_type=jnp.float32)
        + jnp.dot(xu[...], wur[...], preferred_element_type=jnp.float32)
        + bu[...]
    )
    om[...] = jax.nn.relu(
        jnp.dot(mm, wmm[...], preferred_element_type=jnp.float32)
        + jnp.dot(xm[...], wmr[...], preferred_element_type=jnp.float32)
        + bm[...]
    )


def _row_spec(h):
    return pl.BlockSpec((RB, h), lambda i: (i, 0))


def _full_spec(r, c):
    return pl.BlockSpec((r, c), lambda i: (0, 0))


_layer_tc = pl.pallas_call(
    _layer_tc_body,
    grid=(N // RB,),
    in_specs=[
        _row_spec(H),
        _row_spec(H),
        _row_spec(H),
        _row_spec(H),
        _row_spec(H),
        _row_spec(H),
        _full_spec(H, H),
        _full_spec(H, H),
        _full_spec(1, H),
        _full_spec(H, H),
        _full_spec(H, H),
        _full_spec(1, H),
    ],
    out_specs=[_row_spec(H), _row_spec(H)],
    out_shape=[jax.ShapeDtypeStruct((N, H), jnp.float32)] * 2,
)


def _proj_tc_body(
    xu1, xu2, xu3, xm1, xm2, xm3, pw1, pw2, pw3, pbu, qw1, qw2, qw3, pbm,
    c1t, c1btm, pu, pm
):
    ju = (
        jnp.dot(xu1[...], pw1[...], preferred_element_type=jnp.float32)
        + jnp.dot(xu2[...], pw2[...], preferred_element_type=jnp.float32)
        + jnp.dot(xu3[...], pw3[...], preferred_element_type=jnp.float32)
        + pbu[...]
    )
    jm = (
        jnp.dot(xm1[...], qw1[...], preferred_element_type=jnp.float32)
        + jnp.dot(xm2[...], qw2[...], preferred_element_type=jnp.float32)
        + jnp.dot(xm3[...], qw3[...], preferred_element_type=jnp.float32)
        + pbm[...]
    )
    pu[...] = jnp.dot(ju, c1t[...], preferred_element_type=jnp.float32)
    pm[...] = jnp.dot(jm, c1btm[...], preferred_element_type=jnp.float32)


_proj_tc = pl.pallas_call(
    _proj_tc_body,
    grid=(N // RB,),
    in_specs=[_row_spec(H)] * 6
    + [_full_spec(H, H), _full_spec(H, H), _full_spec(H, H), _full_spec(1, H)] * 2
    + [_full_spec(H, H), _full_spec(H, H)],
    out_specs=[_row_spec(H), _row_spec(H)],
    out_shape=[jax.ShapeDtypeStruct((N, H), jnp.float32)] * 2,
)

BR = 1024
_INV_BN = 1.0 / math.sqrt(1.0 + 1e-5)


def _cls_tc_body(z, c1b, g1, b1, w2, c2b, g2, b2, c3w, c3b, out):
    h = z[...] + c1b[...]
    h = jax.nn.relu(g1[...] * h * _INV_BN + b1[...])
    h = jnp.dot(h, w2[...], preferred_element_type=jnp.float32) + c2b[...]
    h = jax.nn.relu(g2[...] * h * _INV_BN + b2[...])
    o = jnp.sum(h * c3w[...], axis=1) + c3b[0]
    out[...] = o.reshape(BR // 128, 128)


_cls_tc = pl.pallas_call(
    _cls_tc_body,
    grid=(L // BR,),
    in_specs=[
        pl.BlockSpec((BR, H), lambda i: (i, 0)),
        _full_spec(1, H),
        _full_spec(1, H),
        _full_spec(1, H),
        _full_spec(H, H // 2),
        _full_spec(1, H // 2),
        _full_spec(1, H // 2),
        _full_spec(1, H // 2),
        _full_spec(1, H // 2),
        pl.BlockSpec(memory_space=pltpu.SMEM),
    ],
    out_specs=pl.BlockSpec((BR // 128, 128), lambda i: (i, 0)),
    out_shape=jax.ShapeDtypeStruct((L // 128, 128), jnp.float32),
)


def kernel(
    user_n_id, movie_x, edge_index_u2m, edge_index_m2u, edge_label_index,
    user_emb,
    Wu0_msg, Wu0_root, bu0, Wm0_msg, Wm0_root, bm0,
    Wu1_msg, Wu1_root, bu1, Wm1_msg, Wm1_root, bm1,
    Wu2_msg, Wu2_root, bu2, Wm2_msg, Wm2_root, bm2,
    proj_u_W, proj_u_b, proj_m_W, proj_m_b,
    c1_W, c1_b, bn1_g, bn1_b, c2_W, c2_b, bn2_g, bn2_b, c3_W, c3_b,
):
    f32 = jnp.float32
    pad = EP - E

    def prep(ei):
        src = jnp.concatenate([ei[0], jnp.zeros((pad,), jnp.int32)])
        dst = jnp.concatenate([ei[1], jnp.full((pad,), N, jnp.int32)])
        return (
            src.reshape(EP // 128, 128),
            dst.reshape(EP // 128, 128),
            src.reshape(EP // EW, EW),
            dst.reshape(EP // EW, EW),
        )

    smu, dmu, smu64, dmu64 = prep(edge_index_m2u)
    sum_, dum, sum64, dum64 = prep(edge_index_u2m)
    e0 = edge_label_index[0].reshape(L // 128, 128)
    e1 = edge_label_index[1].reshape(L // 128, 128)
    zeros = jnp.zeros((NP, H), f32)
    ones = jnp.ones((128, H), f32)

    cnt = _counts(dmu, dum, ones, zeros)
    cu, cm = cnt[0], cnt[1]

    xu = user_emb
    xm = movie_x
    wu = [(Wu0_msg, Wu0_root, bu0), (Wu1_msg, Wu1_root, bu1), (Wu2_msg, Wu2_root, bu2)]
    wm = [(Wm0_msg, Wm0_root, bm0), (Wm1_msg, Wm1_root, bm1), (Wm2_msg, Wm2_root, bm2)]
    outs_u, outs_m = [], []
    for lyr in range(3):
        s = _segsum(xm, xu, smu64, dmu64, sum64, dum64, zeros)
        wum, wur, bu = wu[lyr]
        wmm, wmr, bm = wm[lyr]
        xu, xm = _layer_tc(
            s[0], s[1], cu, cm, xu, xm,
            wum, wur, bu.reshape(1, H), wmm, wmr, bm.reshape(1, H),
        )
        outs_u.append(xu)
        outs_m.append(xm)

    pu, pm = _proj_tc(
        outs_u[0], outs_u[1], outs_u[2], outs_m[0], outs_m[1], outs_m[2],
        proj_u_W[0:H], proj_u_W[H : 2 * H], proj_u_W[2 * H : 3 * H],
        proj_u_b.reshape(1, H),
        proj_m_W[0:H], proj_m_W[H : 2 * H], proj_m_W[2 * H : 3 * H],
        proj_m_b.reshape(1, H),
        c1_W[0:H], c1_W[H : 2 * H],
    )

    z = _pairsum(pu, pm, e0, e1)

    out = _cls_tc(
        z,
        c1_b.reshape(1, H),
        bn1_g.reshape(1, H),
        bn1_b.reshape(1, H),
        c2_W,
        c2_b.reshape(1, H // 2),
        bn2_g.reshape(1, H // 2),
        bn2_b.reshape(1, H // 2),
        c3_W.reshape(1, H // 2),
        c3_b,
    )
    return out.reshape(L)


# speedup vs baseline: 1.0697x; 1.0697x over previous
import math

import jax
import jax.numpy as jnp
from jax import lax
from jax.experimental import pallas as pl
from jax.experimental.pallas import tpu as pltpu
from jax.experimental.pallas import tpu_sc as plsc

H = 128
N = 10000
E = 320000
L = 131072
NC, NS = 2, 16
NW = NC * NS
NP = 10112
SL = NP // NS
RPS = 160
EP = RPS * NS * 128
CW = 16
CH = 16
KB = 2
EW = 128
RW = (L // 128) // NW

_mesh = plsc.VectorSubcoreMesh(
    core_axis_name="c", subcore_axis_name="s", num_cores=NC, num_subcores=NS
)


def _segsum_body(
    xm, xu, smu, dmu, sum_, dum, zeros, out,
    sidx, didx, rows0, rows1, acc, gs0, gs1, ss0, ss1
):
    c = lax.axis_index("c")
    s = lax.axis_index("s")
    rows = [rows0, rows1]
    gsem = [gs0, gs1]
    ssem = [ss0, ss1]

    def run(x_hbm, s_hbm, d_hbm):
        pltpu.sync_copy(zeros.at[pl.ds(s * SL, SL)], acc.at[pl.ds(s * SL, SL)])
        plsc.subcore_barrier()

        def chunk(ch, carry):
            base = pl.multiple_of(s * RPS + ch * CH, 8)
            pltpu.sync_copy(s_hbm.at[pl.ds(base, CH)], sidx)
            pltpu.sync_copy(d_hbm.at[pl.ds(base, CH)], didx)
            gd = [None] * KB
            sd = [None] * KB
            for step in range(CH + 1):
                if step < CH:
                    sl = step % KB
                    if step >= KB:
                        sd[sl].wait()
                    gd[sl] = pltpu.async_copy(
                        x_hbm.at[sidx.at[step]], rows[sl], gsem[sl]
                    )
                prev = step - 1
                if prev >= 0:
                    sp = prev % KB
                    gd[sp].wait()
                    sd[sp] = pltpu.async_copy(
                        rows[sp], acc.at[didx.at[prev]], ssem[sp], add=True
                    )
            for k in range(KB):
                sd[k].wait()
            return carry

        lax.fori_loop(0, RPS // CH, chunk, 0)
        plsc.subcore_barrier()
        pltpu.sync_copy(acc.at[pl.ds(s * SL, SL)], out.at[c, pl.ds(s * SL, SL)])

    @pl.when(c == 0)
    def _():
        run(xm, smu, dmu)

    @pl.when(c == 1)
    def _():
        run(xu, sum_, dum)


_segsum = pl.kernel(
    _segsum_body,
    out_type=jax.ShapeDtypeStruct((NC, NP, H), jnp.float32),
    mesh=_mesh,
    scratch_types=[
        pltpu.VMEM((CH, EW), jnp.int32),
        pltpu.VMEM((CH, EW), jnp.int32),
        pltpu.VMEM((EW, H), jnp.float32),
        pltpu.VMEM((EW, H), jnp.float32),
        pltpu.VMEM_SHARED((NP, H), jnp.float32),
    ]
    + [pltpu.SemaphoreType.DMA] * 4,
)


def _counts_body(dmu, dum, ones, zeros, out, didx, onev, acc, cs0, cs1):
    c = lax.axis_index("c")
    s = lax.axis_index("s")
    csem = [cs0, cs1]
    pltpu.sync_copy(ones, onev)

    def run(d_hbm):
        pltpu.sync_copy(d_hbm.at[pl.ds(s * RPS, RPS)], didx)
        pltpu.sync_copy(zeros.at[pl.ds(s * SL, SL)], acc.at[pl.ds(s * SL, SL)])
        plsc.subcore_barrier()

        def chunk(ch, carry):
            sd = [None] * 2
            for r in range(CH):
                sl = r % 2
                if r >= 2:
                    sd[sl].wait()
                sd[sl] = pltpu.async_copy(
                    onev, acc.at[didx.at[ch * CH + r]], csem[sl], add=True
                )
            for k in range(2):
                sd[k].wait()
            return carry

        lax.fori_loop(0, RPS // CH, chunk, 0)
        plsc.subcore_barrier()
        pltpu.sync_copy(acc.at[pl.ds(s * SL, SL)], out.at[c, pl.ds(s * SL, SL)])

    @pl.when(c == 0)
    def _():
        run(dmu)

    @pl.when(c == 1)
    def _():
        run(dum)


_counts = pl.kernel(
    _counts_body,
    out_type=jax.ShapeDtypeStruct((NC, NP, H), jnp.float32),
    mesh=_mesh,
    scratch_types=[
        pltpu.VMEM((RPS, 128), jnp.int32),
        pltpu.VMEM((128, H), jnp.float32),
        pltpu.VMEM_SHARED((NP, H), jnp.float32),
        pltpu.SemaphoreType.DMA,
        pltpu.SemaphoreType.DMA,
    ],
)


CHP = 8


def _pairsum_body(
    pu, pm, e0, e1, out, idx0, idx1, b0, b1, b2,
    g0, g1, g2, a0, a1, a2, o0, o1, o2
):
    c = lax.axis_index("c")
    s = lax.axis_index("s")
    w = s * NC + c
    base = w * RW
    bufs = [b0, b1, b2]
    gsem = [g0, g1, g2]
    asem = [a0, a1, a2]
    osem = [o0, o1, o2]
    pltpu.sync_copy(e0.at[pl.ds(base, RW)], idx0)
    pltpu.sync_copy(e1.at[pl.ds(base, RW)], idx1)

    def chunk(ch, carry):
        gd = [None] * 3
        ad = [None] * 3
        od = [None] * 3
        for step in range(CHP + 2):
            r0 = step
            if r0 < CHP:
                sl = r0 % 3
                if r0 >= 3:
                    od[sl].wait()
                gd[sl] = pltpu.async_copy(
                    pu.at[idx0.at[ch * CHP + r0]], bufs[sl], gsem[sl]
                )
            r1 = step - 1
            if 0 <= r1 < CHP:
                sl = r1 % 3
                gd[sl].wait()
                ad[sl] = pltpu.async_copy(
                    pm.at[idx1.at[ch * CHP + r1]], bufs[sl], asem[sl], add=True
                )
            r2 = step - 2
            if 0 <= r2 < CHP:
                sl = r2 % 3
                ad[sl].wait()
                od[sl] = pltpu.async_copy(
                    bufs[sl],
                    out.at[pl.ds((base + ch * CHP + r2) * 128, 128)],
                    osem[sl],
                )
        for k in range(3):
            od[k].wait()
        return carry

    lax.fori_loop(0, RW // CHP, chunk, 0)


_pairsum = pl.kernel(
    _pairsum_body,
    out_type=jax.ShapeDtypeStruct((L, H), jnp.float32),
    mesh=_mesh,
    scratch_types=[
        pltpu.VMEM((RW, 128), jnp.int32),
        pltpu.VMEM((RW, 128), jnp.int32),
        pltpu.VMEM((128, H), jnp.float32),
        pltpu.VMEM((128, H), jnp.float32),
        pltpu.VMEM((128, H), jnp.float32),
    ]
    + [pltpu.SemaphoreType.DMA] * 9,
)


RB = 1000


def _layer_tc_body(su, sm, cu, cm, xu, xm, wum, wur, bu, wmm, wmr, bm, ou, om):
    mu = su[...] / jnp.maximum(cu[...][:, :1], 1.0)
    mm = sm[...] / jnp.maximum(cm[...][:, :1], 1.0)
    ou[...] = jax.nn.relu(
        jnp.dot(mu, wum[...], preferred_element_type=jnp.float32)
        + jnp.dot(xu[...], wur[...], preferred_element_type=jnp.float32)
        + bu[...]
    )
    om[...] = jax.nn.relu(
        jnp.dot(mm, wmm[...], preferred_element_type=jnp.float32)
        + jnp.dot(xm[...], wmr[...], preferred_element_type=jnp.float32)
        + bm[...]
    )


def _row_spec(h):
    return pl.BlockSpec((RB, h), lambda i: (i, 0))


def _full_spec(r, c):
    return pl.BlockSpec((r, c), lambda i: (0, 0))


_layer_tc = pl.pallas_call(
    _layer_tc_body,
    grid=(N // RB,),
    in_specs=[
        _row_spec(H),
        _row_spec(H),
        _row_spec(H),
        _row_spec(H),
        _row_spec(H),
        _row_spec(H),
        _full_spec(H, H),
        _full_spec(H, H),
        _full_spec(1, H),
        _full_spec(H, H),
        _full_spec(H, H),
        _full_spec(1, H),
    ],
    out_specs=[_row_spec(H), _row_spec(H)],
    out_shape=[jax.ShapeDtypeStruct((N, H), jnp.float32)] * 2,
)


def _proj_tc_body(
    xu1, xu2, xu3, xm1, xm2, xm3, pw1, pw2, pw3, pbu, qw1, qw2, qw3, pbm,
    c1t, c1btm, pu, pm
):
    ju = (
        jnp.dot(xu1[...], pw1[...], preferred_element_type=jnp.float32)
        + jnp.dot(xu2[...], pw2[...], preferred_element_type=jnp.float32)
        + jnp.dot(xu3[...], pw3[...], preferred_element_type=jnp.float32)
        + pbu[...]
    )
    jm = (
        jnp.dot(xm1[...], qw1[...], preferred_element_type=jnp.float32)
        + jnp.dot(xm2[...], qw2[...], preferred_element_type=jnp.float32)
        + jnp.dot(xm3[...], qw3[...], preferred_element_type=jnp.float32)
        + pbm[...]
    )
    pu[...] = jnp.dot(ju, c1t[...], preferred_element_type=jnp.float32)
    pm[...] = jnp.dot(jm, c1btm[...], preferred_element_type=jnp.float32)


_proj_tc = pl.pallas_call(
    _proj_tc_body,
    grid=(N // RB,),
    in_specs=[_row_spec(H)] * 6
    + [_full_spec(H, H), _full_spec(H, H), _full_spec(H, H), _full_spec(1, H)] * 2
    + [_full_spec(H, H), _full_spec(H, H)],
    out_specs=[_row_spec(H), _row_spec(H)],
    out_shape=[jax.ShapeDtypeStruct((N, H), jnp.float32)] * 2,
)

BR = 1024
_INV_BN = 1.0 / math.sqrt(1.0 + 1e-5)


def _cls_tc_body(z, c1b, g1, b1, w2, c2b, g2, b2, c3w, c3b, out):
    h = z[...] + c1b[...]
    h = jax.nn.relu(g1[...] * h * _INV_BN + b1[...])
    h = jnp.dot(h, w2[...], preferred_element_type=jnp.float32) + c2b[...]
    h = jax.nn.relu(g2[...] * h * _INV_BN + b2[...])
    o = jnp.sum(h * c3w[...], axis=1) + c3b[0]
    out[...] = o.reshape(BR // 128, 128)


_cls_tc = pl.pallas_call(
    _cls_tc_body,
    grid=(L // BR,),
    in_specs=[
        pl.BlockSpec((BR, H), lambda i: (i, 0)),
        _full_spec(1, H),
        _full_spec(1, H),
        _full_spec(1, H),
        _full_spec(H, H // 2),
        _full_spec(1, H // 2),
        _full_spec(1, H // 2),
        _full_spec(1, H // 2),
        _full_spec(1, H // 2),
        pl.BlockSpec(memory_space=pltpu.SMEM),
    ],
    out_specs=pl.BlockSpec((BR // 128, 128), lambda i: (i, 0)),
    out_shape=jax.ShapeDtypeStruct((L // 128, 128), jnp.float32),
)


def kernel(
    user_n_id, movie_x, edge_index_u2m, edge_index_m2u, edge_label_index,
    user_emb,
    Wu0_msg, Wu0_root, bu0, Wm0_msg, Wm0_root, bm0,
    Wu1_msg, Wu1_root, bu1, Wm1_msg, Wm1_root, bm1,
    Wu2_msg, Wu2_root, bu2, Wm2_msg, Wm2_root, bm2,
    proj_u_W, proj_u_b, proj_m_W, proj_m_b,
    c1_W, c1_b, bn1_g, bn1_b, c2_W, c2_b, bn2_g, bn2_b, c3_W, c3_b,
):
    f32 = jnp.float32
    pad = EP - E

    def prep(ei):
        src = jnp.concatenate([ei[0], jnp.zeros((pad,), jnp.int32)])
        dst = jnp.concatenate([ei[1], jnp.full((pad,), N, jnp.int32)])
        return src.reshape(EP // 128, 128), dst.reshape(EP // 128, 128)

    smu, dmu = prep(edge_index_m2u)
    sum_, dum = prep(edge_index_u2m)
    e0 = edge_label_index[0].reshape(L // 128, 128)
    e1 = edge_label_index[1].reshape(L // 128, 128)
    zeros = jnp.zeros((NP, H), f32)
    ones = jnp.ones((128, H), f32)

    cnt = _counts(dmu, dum, ones, zeros)
    cu, cm = cnt[0], cnt[1]

    xu = user_emb
    xm = movie_x
    wu = [(Wu0_msg, Wu0_root, bu0), (Wu1_msg, Wu1_root, bu1), (Wu2_msg, Wu2_root, bu2)]
    wm = [(Wm0_msg, Wm0_root, bm0), (Wm1_msg, Wm1_root, bm1), (Wm2_msg, Wm2_root, bm2)]
    outs_u, outs_m = [], []
    for lyr in range(3):
        s = _segsum(xm, xu, smu, dmu, sum_, dum, zeros)
        wum, wur, bu = wu[lyr]
        wmm, wmr, bm = wm[lyr]
        xu, xm = _layer_tc(
            s[0], s[1], cu, cm, xu, xm,
            wum, wur, bu.reshape(1, H), wmm, wmr, bm.reshape(1, H),
        )
        outs_u.append(xu)
        outs_m.append(xm)

    pu, pm = _proj_tc(
        outs_u[0], outs_u[1], outs_u[2], outs_m[0], outs_m[1], outs_m[2],
        proj_u_W[0:H], proj_u_W[H : 2 * H], proj_u_W[2 * H : 3 * H],
        proj_u_b.reshape(1, H),
        proj_m_W[0:H], proj_m_W[H : 2 * H], proj_m_W[2 * H : 3 * H],
        proj_m_b.reshape(1, H),
        c1_W[0:H], c1_W[H : 2 * H],
    )

    z = _pairsum(pu, pm, e0, e1)

    out = _cls_tc(
        z,
        c1_b.reshape(1, H),
        bn1_g.reshape(1, H),
        bn1_b.reshape(1, H),
        c2_W,
        c2_b.reshape(1, H // 2),
        bn2_g.reshape(1, H // 2),
        bn2_b.reshape(1, H // 2),
        c3_W.reshape(1, H // 2),
        c3_b,
    )
    return out.reshape(L)


# speedup vs baseline: 1.0740x; 1.0041x over previous
import math

import jax
import jax.numpy as jnp
from jax import lax
from jax.experimental import pallas as pl
from jax.experimental.pallas import tpu as pltpu
from jax.experimental.pallas import tpu_sc as plsc

H = 128
N = 10000
E = 320000
L = 131072
NC, NS = 2, 16
NW = NC * NS
NP = 10112
SL = NP // NS
RPS = 160
EP = RPS * NS * 128
CW = 16
CH = 16
KB = 2
EW = 128
RW = (L // 128) // NW

_mesh = plsc.VectorSubcoreMesh(
    core_axis_name="c", subcore_axis_name="s", num_cores=NC, num_subcores=NS
)


def _segsum_body(
    xm, xu, smu, dmu, sum_, dum, zeros, out,
    sidx, didx, rows0, rows1, acc, gs0, gs1, ss0, ss1
):
    c = lax.axis_index("c")
    s = lax.axis_index("s")
    rows = [rows0, rows1]
    gsem = [gs0, gs1]
    ssem = [ss0, ss1]

    def run(x_hbm, s_hbm, d_hbm):
        pltpu.sync_copy(zeros.at[pl.ds(s * SL, SL)], acc.at[pl.ds(s * SL, SL)])
        plsc.subcore_barrier()

        def chunk(ch, carry):
            base = pl.multiple_of(s * RPS + ch * CH, 8)
            pltpu.sync_copy(s_hbm.at[pl.ds(base, CH)], sidx)
            pltpu.sync_copy(d_hbm.at[pl.ds(base, CH)], didx)
            gd = [None] * KB
            sd = [None] * KB
            for step in range(CH + 1):
                if step < CH:
                    sl = step % KB
                    if step >= KB:
                        sd[sl].wait()
                    gd[sl] = pltpu.async_copy(
                        x_hbm.at[sidx.at[step]], rows[sl], gsem[sl]
                    )
                prev = step - 1
                if prev >= 0:
                    sp = prev % KB
                    gd[sp].wait()
                    sd[sp] = pltpu.async_copy(
                        rows[sp], acc.at[didx.at[prev]], ssem[sp], add=True
                    )
            for k in range(KB):
                sd[k].wait()
            return carry

        lax.fori_loop(0, RPS // CH, chunk, 0)
        plsc.subcore_barrier()
        pltpu.sync_copy(acc.at[pl.ds(s * SL, SL)], out.at[c, pl.ds(s * SL, SL)])

    @pl.when(c == 0)
    def _():
        run(xm, smu, dmu)

    @pl.when(c == 1)
    def _():
        run(xu, sum_, dum)


_segsum = pl.kernel(
    _segsum_body,
    out_type=jax.ShapeDtypeStruct((NC, NP, H), jnp.float32),
    mesh=_mesh,
    scratch_types=[
        pltpu.VMEM((CH, EW), jnp.int32),
        pltpu.VMEM((CH, EW), jnp.int32),
        pltpu.VMEM((EW, H), jnp.float32),
        pltpu.VMEM((EW, H), jnp.float32),
        pltpu.VMEM_SHARED((NP, H), jnp.float32),
    ]
    + [pltpu.SemaphoreType.DMA] * 4,
)


def _counts_body(dmu, dum, ones, zeros, out, didx, onev, acc, cs0, cs1):
    c = lax.axis_index("c")
    s = lax.axis_index("s")
    csem = [cs0, cs1]
    pltpu.sync_copy(ones, onev)

    def run(d_hbm):
        pltpu.sync_copy(d_hbm.at[pl.ds(s * RPS, RPS)], didx)
        pltpu.sync_copy(zeros.at[pl.ds(s * SL, SL)], acc.at[pl.ds(s * SL, SL)])
        plsc.subcore_barrier()

        def chunk(ch, carry):
            sd = [None] * 2
            for r in range(CH):
                sl = r % 2
                if r >= 2:
                    sd[sl].wait()
                sd[sl] = pltpu.async_copy(
                    onev, acc.at[didx.at[ch * CH + r]], csem[sl], add=True
                )
            for k in range(2):
                sd[k].wait()
            return carry

        lax.fori_loop(0, RPS // CH, chunk, 0)
        plsc.subcore_barrier()
        pltpu.sync_copy(acc.at[pl.ds(s * SL, SL)], out.at[c, pl.ds(s * SL, SL)])

    @pl.when(c == 0)
    def _():
        run(dmu)

    @pl.when(c == 1)
    def _():
        run(dum)


_counts = pl.kernel(
    _counts_body,
    out_type=jax.ShapeDtypeStruct((NC, NP, H), jnp.float32),
    mesh=_mesh,
    scratch_types=[
        pltpu.VMEM((RPS, 128), jnp.int32),
        pltpu.VMEM((128, H), jnp.float32),
        pltpu.VMEM_SHARED((NP, H), jnp.float32),
        pltpu.SemaphoreType.DMA,
        pltpu.SemaphoreType.DMA,
    ],
)


CHP = 8


def _pairsum_body(
    pu, pm, e0, e1, out, idx0, idx1, b0, b1, b2,
    g0, g1, g2, a0, a1, a2, o0, o1, o2
):
    c = lax.axis_index("c")
    s = lax.axis_index("s")
    w = s * NC + c
    base = w * RW
    bufs = [b0, b1, b2]
    gsem = [g0, g1, g2]
    asem = [a0, a1, a2]
    osem = [o0, o1, o2]
    pltpu.sync_copy(e0.at[pl.ds(base, RW)], idx0)
    pltpu.sync_copy(e1.at[pl.ds(base, RW)], idx1)

    def chunk(ch, carry):
        gd = [None] * 3
        ad = [None] * 3
        od = [None] * 3
        for step in range(CHP + 2):
            r0 = step
            if r0 < CHP:
                sl = r0 % 3
                if r0 >= 3:
                    od[sl].wait()
                gd[sl] = pltpu.async_copy(
                    pu.at[idx0.at[ch * CHP + r0]], bufs[sl], gsem[sl]
                )
            r1 = step - 1
            if 0 <= r1 < CHP:
                sl = r1 % 3
                gd[sl].wait()
                ad[sl] = pltpu.async_copy(
                    pm.at[idx1.at[ch * CHP + r1]], bufs[sl], asem[sl], add=True
                )
            r2 = step - 2
            if 0 <= r2 < CHP:
                sl = r2 % 3
                ad[sl].wait()
                od[sl] = pltpu.async_copy(
                    bufs[sl],
                    out.at[pl.ds((base + ch * CHP + r2) * 128, 128)],
                    osem[sl],
                )
        for k in range(3):
            od[k].wait()
        return carry

    lax.fori_loop(0, RW // CHP, chunk, 0)


_pairsum = pl.kernel(
    _pairsum_body,
    out_type=jax.ShapeDtypeStruct((L, H), jnp.float32),
    mesh=_mesh,
    scratch_types=[
        pltpu.VMEM((RW, 128), jnp.int32),
        pltpu.VMEM((RW, 128), jnp.int32),
        pltpu.VMEM((128, H), jnp.float32),
        pltpu.VMEM((128, H), jnp.float32),
        pltpu.VMEM((128, H), jnp.float32),
    ]
    + [pltpu.SemaphoreType.DMA] * 9,
)


RB = 1000


def _layer_tc_body(s2, cnt2, xu, xm, wum, wur, bu, wmm, wmr, bm, ou, om):
    su, sm = s2[...][0], s2[...][1]
    cu, cm = cnt2[...][0], cnt2[...][1]
    mu = su / jnp.maximum(cu[:, :1], 1.0)
    mm = sm / jnp.maximum(cm[:, :1], 1.0)
    ou[...] = jax.nn.relu(
        jnp.dot(mu, wum[...], preferred_element_type=jnp.float32)
        + jnp.dot(xu[...], wur[...], preferred_element_type=jnp.float32)
        + bu[...]
    )
    om[...] = jax.nn.relu(
        jnp.dot(mm, wmm[...], preferred_element_type=jnp.float32)
        + jnp.dot(xm[...], wmr[...], preferred_element_type=jnp.float32)
        + bm[...]
    )


def _row_spec(h):
    return pl.BlockSpec((RB, h), lambda i: (i, 0))


def _full_spec(r, c):
    return pl.BlockSpec((r, c), lambda i: (0, 0))


_layer_tc = pl.pallas_call(
    _layer_tc_body,
    grid=(N // RB,),
    in_specs=[
        pl.BlockSpec((NC, RB, H), lambda i: (0, i, 0)),
        pl.BlockSpec((NC, RB, H), lambda i: (0, i, 0)),
        _row_spec(H),
        _row_spec(H),
        _full_spec(H, H),
        _full_spec(H, H),
        _full_spec(1, H),
        _full_spec(H, H),
        _full_spec(H, H),
        _full_spec(1, H),
    ],
    out_specs=[_row_spec(H), _row_spec(H)],
    out_shape=[jax.ShapeDtypeStruct((N, H), jnp.float32)] * 2,
)


def _proj_tc_body(
    xu1, xu2, xu3, xm1, xm2, xm3, pw1, pw2, pw3, pbu, qw1, qw2, qw3, pbm,
    c1t, c1btm, pu, pm
):
    ju = (
        jnp.dot(xu1[...], pw1[...], preferred_element_type=jnp.float32)
        + jnp.dot(xu2[...], pw2[...], preferred_element_type=jnp.float32)
        + jnp.dot(xu3[...], pw3[...], preferred_element_type=jnp.float32)
        + pbu[...]
    )
    jm = (
        jnp.dot(xm1[...], qw1[...], preferred_element_type=jnp.float32)
        + jnp.dot(xm2[...], qw2[...], preferred_element_type=jnp.float32)
        + jnp.dot(xm3[...], qw3[...], preferred_element_type=jnp.float32)
        + pbm[...]
    )
    pu[...] = jnp.dot(ju, c1t[...], preferred_element_type=jnp.float32)
    pm[...] = jnp.dot(jm, c1btm[...], preferred_element_type=jnp.float32)


_proj_tc = pl.pallas_call(
    _proj_tc_body,
    grid=(N // RB,),
    in_specs=[_row_spec(H)] * 6
    + [_full_spec(H, H), _full_spec(H, H), _full_spec(H, H), _full_spec(1, H)] * 2
    + [_full_spec(H, H), _full_spec(H, H)],
    out_specs=[_row_spec(H), _row_spec(H)],
    out_shape=[jax.ShapeDtypeStruct((N, H), jnp.float32)] * 2,
)

BR = 1024
_INV_BN = 1.0 / math.sqrt(1.0 + 1e-5)


def _cls_tc_body(z, c1b, g1, b1, w2, c2b, g2, b2, c3w, c3b, out):
    h = z[...] + c1b[...]
    h = jax.nn.relu(g1[...] * h * _INV_BN + b1[...])
    h = jnp.dot(h, w2[...], preferred_element_type=jnp.float32) + c2b[...]
    h = jax.nn.relu(g2[...] * h * _INV_BN + b2[...])
    o = jnp.sum(h * c3w[...], axis=1) + c3b[0]
    out[...] = o.reshape(BR // 128, 128)


_cls_tc = pl.pallas_call(
    _cls_tc_body,
    grid=(L // BR,),
    in_specs=[
        pl.BlockSpec((BR, H), lambda i: (i, 0)),
        _full_spec(1, H),
        _full_spec(1, H),
        _full_spec(1, H),
        _full_spec(H, H // 2),
        _full_spec(1, H // 2),
        _full_spec(1, H // 2),
        _full_spec(1, H // 2),
        _full_spec(1, H // 2),
        pl.BlockSpec(memory_space=pltpu.SMEM),
    ],
    out_specs=pl.BlockSpec((BR // 128, 128), lambda i: (i, 0)),
    out_shape=jax.ShapeDtypeStruct((L // 128, 128), jnp.float32),
)


def kernel(
    user_n_id, movie_x, edge_index_u2m, edge_index_m2u, edge_label_index,
    user_emb,
    Wu0_msg, Wu0_root, bu0, Wm0_msg, Wm0_root, bm0,
    Wu1_msg, Wu1_root, bu1, Wm1_msg, Wm1_root, bm1,
    Wu2_msg, Wu2_root, bu2, Wm2_msg, Wm2_root, bm2,
    proj_u_W, proj_u_b, proj_m_W, proj_m_b,
    c1_W, c1_b, bn1_g, bn1_b, c2_W, c2_b, bn2_g, bn2_b, c3_W, c3_b,
):
    f32 = jnp.float32
    pad = EP - E

    def prep(ei):
        src = jnp.concatenate([ei[0], jnp.zeros((pad,), jnp.int32)])
        dst = jnp.concatenate([ei[1], jnp.full((pad,), N, jnp.int32)])
        return src.reshape(EP // 128, 128), dst.reshape(EP // 128, 128)

    smu, dmu = prep(edge_index_m2u)
    sum_, dum = prep(edge_index_u2m)
    e0 = edge_label_index[0].reshape(L // 128, 128)
    e1 = edge_label_index[1].reshape(L // 128, 128)
    zeros = jnp.zeros((NP, H), f32)
    ones = jnp.ones((128, H), f32)

    cnt = _counts(dmu, dum, ones, zeros)

    xu = user_emb
    xm = movie_x
    wu = [(Wu0_msg, Wu0_root, bu0), (Wu1_msg, Wu1_root, bu1), (Wu2_msg, Wu2_root, bu2)]
    wm = [(Wm0_msg, Wm0_root, bm0), (Wm1_msg, Wm1_root, bm1), (Wm2_msg, Wm2_root, bm2)]
    outs_u, outs_m = [], []
    for lyr in range(3):
        s = _segsum(xm, xu, smu, dmu, sum_, dum, zeros)
        wum, wur, bu = wu[lyr]
        wmm, wmr, bm = wm[lyr]
        xu, xm = _layer_tc(
            s, cnt, xu, xm,
            wum, wur, bu.reshape(1, H), wmm, wmr, bm.reshape(1, H),
        )
        outs_u.append(xu)
        outs_m.append(xm)

    pu, pm = _proj_tc(
        outs_u[0], outs_u[1], outs_u[2], outs_m[0], outs_m[1], outs_m[2],
        proj_u_W[0:H], proj_u_W[H : 2 * H], proj_u_W[2 * H : 3 * H],
        proj_u_b.reshape(1, H),
        proj_m_W[0:H], proj_m_W[H : 2 * H], proj_m_W[2 * H : 3 * H],
        proj_m_b.reshape(1, H),
        c1_W[0:H], c1_W[H : 2 * H],
    )

    z = _pairsum(pu, pm, e0, e1)

    out = _cls_tc(
        z,
        c1_b.reshape(1, H),
        bn1_g.reshape(1, H),
        bn1_b.reshape(1, H),
        c2_W,
        c2_b.reshape(1, H // 2),
        bn2_g.reshape(1, H // 2),
        bn2_b.reshape(1, H // 2),
        c3_W.reshape(1, H // 2),
        c3_b,
    )
    return out.reshape(L)


# speedup vs baseline: 1.0817x; 1.0072x over previous
import math

import jax
import jax.numpy as jnp
from jax import lax
from jax.experimental import pallas as pl
from jax.experimental.pallas import tpu as pltpu
from jax.experimental.pallas import tpu_sc as plsc

H = 128
N = 10000
E = 320000
L = 131072
NC, NS = 2, 16
NW = NC * NS
NP = 10112
SL = NP // NS
RPS = 160
EP = RPS * NS * 128
CW = 16
CH = 16
KB = 2
EW = 128
RW = (L // 128) // NW

_mesh = plsc.VectorSubcoreMesh(
    core_axis_name="c", subcore_axis_name="s", num_cores=NC, num_subcores=NS
)


def _segsum_body(
    xm, xu, smu, dmu, sum_, dum, zeros, out,
    sidx, didx, rows0, rows1, acc, gs0, gs1, ss0, ss1
):
    c = lax.axis_index("c")
    s = lax.axis_index("s")
    rows = [rows0, rows1]
    gsem = [gs0, gs1]
    ssem = [ss0, ss1]

    def run(x_hbm, s_hbm, d_hbm):
        pltpu.sync_copy(zeros.at[pl.ds(s * SL, SL)], acc.at[pl.ds(s * SL, SL)])
        plsc.subcore_barrier()

        def drain_scatter(sl):
            pltpu.make_async_copy(zeros.at[pl.ds(0, EW)], rows[sl], ssem[sl]).wait()

        def chunk(ch, carry):
            par = lax.rem(ch, 2)
            base = pl.multiple_of(s * RPS + ch * CH, 8)
            pltpu.sync_copy(s_hbm.at[pl.ds(base, CH)], sidx.at[par])
            pltpu.sync_copy(d_hbm.at[pl.ds(base, CH)], didx.at[par])
            gd = [None] * KB
            sd = [None] * KB
            for step in range(CH + 1):
                if step < CH:
                    sl = step % KB
                    if step >= KB:
                        sd[sl].wait()
                    else:
                        @pl.when(ch > 0)
                        def _():
                            drain_scatter(sl)
                    gd[sl] = pltpu.async_copy(
                        x_hbm.at[sidx.at[par, step]], rows[sl], gsem[sl]
                    )
                prev = step - 1
                if prev >= 0:
                    sp = prev % KB
                    gd[sp].wait()
                    sd[sp] = pltpu.async_copy(
                        rows[sp], acc.at[didx.at[par, prev]], ssem[sp], add=True
                    )
            return carry

        lax.fori_loop(0, RPS // CH, chunk, 0)
        for k in range(KB):
            drain_scatter(k)
        plsc.subcore_barrier()
        pltpu.sync_copy(acc.at[pl.ds(s * SL, SL)], out.at[c, pl.ds(s * SL, SL)])

    @pl.when(c == 0)
    def _():
        run(xm, smu, dmu)

    @pl.when(c == 1)
    def _():
        run(xu, sum_, dum)


_segsum = pl.kernel(
    _segsum_body,
    out_type=jax.ShapeDtypeStruct((NC, NP, H), jnp.float32),
    mesh=_mesh,
    scratch_types=[
        pltpu.VMEM((2, CH, EW), jnp.int32),
        pltpu.VMEM((2, CH, EW), jnp.int32),
        pltpu.VMEM((EW, H), jnp.float32),
        pltpu.VMEM((EW, H), jnp.float32),
        pltpu.VMEM_SHARED((NP, H), jnp.float32),
    ]
    + [pltpu.SemaphoreType.DMA] * 4,
)


def _counts_body(dmu, dum, ones, zeros, out, didx, onev, acc, cs0, cs1):
    c = lax.axis_index("c")
    s = lax.axis_index("s")
    csem = [cs0, cs1]
    pltpu.sync_copy(ones, onev)

    def run(d_hbm):
        pltpu.sync_copy(d_hbm.at[pl.ds(s * RPS, RPS)], didx)
        pltpu.sync_copy(zeros.at[pl.ds(s * SL, SL)], acc.at[pl.ds(s * SL, SL)])
        plsc.subcore_barrier()

        def chunk(ch, carry):
            sd = [None] * 2
            for r in range(CH):
                sl = r % 2
                if r >= 2:
                    sd[sl].wait()
                sd[sl] = pltpu.async_copy(
                    onev, acc.at[didx.at[ch * CH + r]], csem[sl], add=True
                )
            for k in range(2):
                sd[k].wait()
            return carry

        lax.fori_loop(0, RPS // CH, chunk, 0)
        plsc.subcore_barrier()
        pltpu.sync_copy(acc.at[pl.ds(s * SL, SL)], out.at[c, pl.ds(s * SL, SL)])

    @pl.when(c == 0)
    def _():
        run(dmu)

    @pl.when(c == 1)
    def _():
        run(dum)


_counts = pl.kernel(
    _counts_body,
    out_type=jax.ShapeDtypeStruct((NC, NP, H), jnp.float32),
    mesh=_mesh,
    scratch_types=[
        pltpu.VMEM((RPS, 128), jnp.int32),
        pltpu.VMEM((128, H), jnp.float32),
        pltpu.VMEM_SHARED((NP, H), jnp.float32),
        pltpu.SemaphoreType.DMA,
        pltpu.SemaphoreType.DMA,
    ],
)


CHP = 8


def _pairsum_body(
    pu, pm, e0, e1, out, idx0, idx1, b0, b1, b2,
    g0, g1, g2, a0, a1, a2, o0, o1, o2
):
    c = lax.axis_index("c")
    s = lax.axis_index("s")
    w = s * NC + c
    base = w * RW
    bufs = [b0, b1, b2]
    gsem = [g0, g1, g2]
    asem = [a0, a1, a2]
    osem = [o0, o1, o2]
    pltpu.sync_copy(e0.at[pl.ds(base, RW)], idx0)
    pltpu.sync_copy(e1.at[pl.ds(base, RW)], idx1)

    def chunk(ch, carry):
        gd = [None] * 3
        ad = [None] * 3
        od = [None] * 3
        for step in range(CHP + 2):
            r0 = step
            if r0 < CHP:
                sl = r0 % 3
                if r0 >= 3:
                    od[sl].wait()
                gd[sl] = pltpu.async_copy(
                    pu.at[idx0.at[ch * CHP + r0]], bufs[sl], gsem[sl]
                )
            r1 = step - 1
            if 0 <= r1 < CHP:
                sl = r1 % 3
                gd[sl].wait()
                ad[sl] = pltpu.async_copy(
                    pm.at[idx1.at[ch * CHP + r1]], bufs[sl], asem[sl], add=True
                )
            r2 = step - 2
            if 0 <= r2 < CHP:
                sl = r2 % 3
                ad[sl].wait()
                od[sl] = pltpu.async_copy(
                    bufs[sl],
                    out.at[pl.ds((base + ch * CHP + r2) * 128, 128)],
                    osem[sl],
                )
        for k in range(3):
            od[k].wait()
        return carry

    lax.fori_loop(0, RW // CHP, chunk, 0)


_pairsum = pl.kernel(
    _pairsum_body,
    out_type=jax.ShapeDtypeStruct((L, H), jnp.float32),
    mesh=_mesh,
    scratch_types=[
        pltpu.VMEM((RW, 128), jnp.int32),
        pltpu.VMEM((RW, 128), jnp.int32),
        pltpu.VMEM((128, H), jnp.float32),
        pltpu.VMEM((128, H), jnp.float32),
        pltpu.VMEM((128, H), jnp.float32),
    ]
    + [pltpu.SemaphoreType.DMA] * 9,
)


RB = 1000


def _layer_tc_body(s2, cnt2, xu, xm, wum, wur, bu, wmm, wmr, bm, ou, om):
    su, sm = s2[...][0], s2[...][1]
    cu, cm = cnt2[...][0], cnt2[...][1]
    mu = su / jnp.maximum(cu[:, :1], 1.0)
    mm = sm / jnp.maximum(cm[:, :1], 1.0)
    ou[...] = jax.nn.relu(
        jnp.dot(mu, wum[...], preferred_element_type=jnp.float32)
        + jnp.dot(xu[...], wur[...], preferred_element_type=jnp.float32)
        + bu[...]
    )
    om[...] = jax.nn.relu(
        jnp.dot(mm, wmm[...], preferred_element_type=jnp.float32)
        + jnp.dot(xm[...], wmr[...], preferred_element_type=jnp.float32)
        + bm[...]
    )


def _row_spec(h):
    return pl.BlockSpec((RB, h), lambda i: (i, 0))


def _full_spec(r, c):
    return pl.BlockSpec((r, c), lambda i: (0, 0))


_layer_tc = pl.pallas_call(
    _layer_tc_body,
    grid=(N // RB,),
    in_specs=[
        pl.BlockSpec((NC, RB, H), lambda i: (0, i, 0)),
        pl.BlockSpec((NC, RB, H), lambda i: (0, i, 0)),
        _row_spec(H),
        _row_spec(H),
        _full_spec(H, H),
        _full_spec(H, H),
        _full_spec(1, H),
        _full_spec(H, H),
        _full_spec(H, H),
        _full_spec(1, H),
    ],
    out_specs=[_row_spec(H), _row_spec(H)],
    out_shape=[jax.ShapeDtypeStruct((N, H), jnp.float32)] * 2,
)


def _proj_tc_body(
    xu1, xu2, xu3, xm1, xm2, xm3, pw1, pw2, pw3, pbu, qw1, qw2, qw3, pbm,
    c1t, c1btm, pu, pm
):
    ju = (
        jnp.dot(xu1[...], pw1[...], preferred_element_type=jnp.float32)
        + jnp.dot(xu2[...], pw2[...], preferred_element_type=jnp.float32)
        + jnp.dot(xu3[...], pw3[...], preferred_element_type=jnp.float32)
        + pbu[...]
    )
    jm = (
        jnp.dot(xm1[...], qw1[...], preferred_element_type=jnp.float32)
        + jnp.dot(xm2[...], qw2[...], preferred_element_type=jnp.float32)
        + jnp.dot(xm3[...], qw3[...], preferred_element_type=jnp.float32)
        + pbm[...]
    )
    pu[...] = jnp.dot(ju, c1t[...], preferred_element_type=jnp.float32)
    pm[...] = jnp.dot(jm, c1btm[...], preferred_element_type=jnp.float32)


_proj_tc = pl.pallas_call(
    _proj_tc_body,
    grid=(N // RB,),
    in_specs=[_row_spec(H)] * 6
    + [_full_spec(H, H), _full_spec(H, H), _full_spec(H, H), _full_spec(1, H)] * 2
    + [_full_spec(H, H), _full_spec(H, H)],
    out_specs=[_row_spec(H), _row_spec(H)],
    out_shape=[jax.ShapeDtypeStruct((N, H), jnp.float32)] * 2,
)

BR = 1024
_INV_BN = 1.0 / math.sqrt(1.0 + 1e-5)


def _cls_tc_body(z, c1b, g1, b1, w2, c2b, g2, b2, c3w, c3b, out):
    h = z[...] + c1b[...]
    h = jax.nn.relu(g1[...] * h * _INV_BN + b1[...])
    h = jnp.dot(h, w2[...], preferred_element_type=jnp.float32) + c2b[...]
    h = jax.nn.relu(g2[...] * h * _INV_BN + b2[...])
    o = jnp.sum(h * c3w[...], axis=1) + c3b[0]
    out[...] = o.reshape(BR // 128, 128)


_cls_tc = pl.pallas_call(
    _cls_tc_body,
    grid=(L // BR,),
    in_specs=[
        pl.BlockSpec((BR, H), lambda i: (i, 0)),
        _full_spec(1, H),
        _full_spec(1, H),
        _full_spec(1, H),
        _full_spec(H, H // 2),
        _full_spec(1, H // 2),
        _full_spec(1, H // 2),
        _full_spec(1, H // 2),
        _full_spec(1, H // 2),
        pl.BlockSpec(memory_space=pltpu.SMEM),
    ],
    out_specs=pl.BlockSpec((BR // 128, 128), lambda i: (i, 0)),
    out_shape=jax.ShapeDtypeStruct((L // 128, 128), jnp.float32),
)


def kernel(
    user_n_id, movie_x, edge_index_u2m, edge_index_m2u, edge_label_index,
    user_emb,
    Wu0_msg, Wu0_root, bu0, Wm0_msg, Wm0_root, bm0,
    Wu1_msg, Wu1_root, bu1, Wm1_msg, Wm1_root, bm1,
    Wu2_msg, Wu2_root, bu2, Wm2_msg, Wm2_root, bm2,
    proj_u_W, proj_u_b, proj_m_W, proj_m_b,
    c1_W, c1_b, bn1_g, bn1_b, c2_W, c2_b, bn2_g, bn2_b, c3_W, c3_b,
):
    f32 = jnp.float32
    pad = EP - E

    def prep(ei):
        src = jnp.concatenate([ei[0], jnp.zeros((pad,), jnp.int32)])
        dst = jnp.concatenate([ei[1], jnp.full((pad,), N, jnp.int32)])
        return src.reshape(EP // 128, 128), dst.reshape(EP // 128, 128)

    smu, dmu = prep(edge_index_m2u)
    sum_, dum = prep(edge_index_u2m)
    e0 = edge_label_index[0].reshape(L // 128, 128)
    e1 = edge_label_index[1].reshape(L // 128, 128)
    zeros = jnp.zeros((NP, H), f32)
    ones = jnp.ones((128, H), f32)

    cnt = _counts(dmu, dum, ones, zeros)

    xu = user_emb
    xm = movie_x
    wu = [(Wu0_msg, Wu0_root, bu0), (Wu1_msg, Wu1_root, bu1), (Wu2_msg, Wu2_root, bu2)]
    wm = [(Wm0_msg, Wm0_root, bm0), (Wm1_msg, Wm1_root, bm1), (Wm2_msg, Wm2_root, bm2)]
    outs_u, outs_m = [], []
    for lyr in range(3):
        s = _segsum(xm, xu, smu, dmu, sum_, dum, zeros)
        wum, wur, bu = wu[lyr]
        wmm, wmr, bm = wm[lyr]
        xu, xm = _layer_tc(
            s, cnt, xu, xm,
            wum, wur, bu.reshape(1, H), wmm, wmr, bm.reshape(1, H),
        )
        outs_u.append(xu)
        outs_m.append(xm)

    pu, pm = _proj_tc(
        outs_u[0], outs_u[1], outs_u[2], outs_m[0], outs_m[1], outs_m[2],
        proj_u_W[0:H], proj_u_W[H : 2 * H], proj_u_W[2 * H : 3 * H],
        proj_u_b.reshape(1, H),
        proj_m_W[0:H], proj_m_W[H : 2 * H], proj_m_W[2 * H : 3 * H],
        proj_m_b.reshape(1, H),
        c1_W[0:H], c1_W[H : 2 * H],
    )

    z = _pairsum(pu, pm, e0, e1)

    out = _cls_tc(
        z,
        c1_b.reshape(1, H),
        bn1_g.reshape(1, H),
        bn1_b.reshape(1, H),
        c2_W,
        c2_b.reshape(1, H // 2),
        bn2_g.reshape(1, H // 2),
        bn2_b.reshape(1, H // 2),
        c3_W.reshape(1, H // 2),
        c3_b,
    )
    return out.reshape(L)
